# baseline (speedup 1.0000x reference)
import jax
import jax.numpy as jnp
from jax.experimental import pallas as pl
from jax.experimental.pallas import tpu as pltpu

_B, _C_IN, _C_OUT, _H, _W = 64, 384, 384, 24, 24
_HW = _H * _W
_E, _K = 8, 2

_ROWS = 8
_NG = _B // _ROWS
_S = 8
_RS = _ROWS // _S
_XB = 3
_OB = 3


def _moe_body(idx_ref, w_ref, x_hbm, WcT_ref, bc_ref, out_hbm,
              xb, ob, in_sems, out_sems):
    g = pl.program_id(0)

    def in_copy(gg, s):
        buf = jax.lax.rem(gg, _XB)
        return pltpu.make_async_copy(
            x_hbm.at[pl.ds(gg * _ROWS + s * _RS, _RS)],
            xb.at[buf, pl.ds(s * _RS, _RS)],
            in_sems.at[buf, s],
        )

    def out_copy(gg, s):
        buf = jax.lax.rem(gg, _OB)
        return pltpu.make_async_copy(
            ob.at[buf, pl.ds(s * _RS, _RS)],
            out_hbm.at[pl.ds(gg * _ROWS + s * _RS, _RS)],
            out_sems.at[buf, s],
        )

    @pl.when(g == 0)
    def _():
        for gg in range(_XB - 1):
            for s in range(_S):
                in_copy(gg, s).start()

    @pl.when(g < _NG - (_XB - 1))
    def _():
        for s in range(_S):
            in_copy(g + (_XB - 1), s).start()

    for s in range(_S):
        in_copy(g, s).wait()

    @pl.when(g >= _OB)
    def _():
        for s in range(_S):
            out_copy(g - _OB, s).wait()

    buf = jax.lax.rem(g, _XB)
    obuf = jax.lax.rem(g, _OB)
    for r in range(_ROWS):
        b = g * _ROWS + r
        i0 = idx_ref[0, b]
        i1 = idx_ref[1, b]
        w0 = w_ref[0, b]
        w1 = w_ref[1, b]
        W_eff = w0 * WcT_ref[i0] + w1 * WcT_ref[i1]
        out = jax.lax.dot_general(
            xb[buf, r], W_eff,
            (((1,), (1,)), ((), ())),
            preferred_element_type=jnp.float32)
        b_row = (w0 * bc_ref[pl.ds(i0, 1), :]
                 + w1 * bc_ref[pl.ds(i1, 1), :])
        ob[obuf, r] = out + b_row

    for s in range(_S):
        out_copy(g, s).start()

    @pl.when(g == _NG - 1)
    def _():
        for gg in range(_NG - _OB, _NG):
            for s in range(_S):
                out_copy(gg, s).wait()


def kernel(x, weights, indices, Wc, bc):
    xt = x.transpose(0, 2, 3, 1).reshape(_B, _HW, _C_IN)
    idx = indices.astype(jnp.int32).T
    w = weights.astype(jnp.float32).T
    WcT = Wc

    grid_spec = pltpu.PrefetchScalarGridSpec(
        num_scalar_prefetch=2,
        grid=(_NG,),
        in_specs=[
            pl.BlockSpec(memory_space=pltpu.MemorySpace.HBM),
            pl.BlockSpec((_E, _C_OUT, _C_IN), lambda b, *_: (0, 0, 0)),
            pl.BlockSpec((_E, _C_OUT), lambda b, *_: (0, 0)),
        ],
        out_specs=pl.BlockSpec(memory_space=pltpu.MemorySpace.HBM),
        scratch_shapes=[
            pltpu.VMEM((_XB, _ROWS, _HW, _C_IN), jnp.float32),
            pltpu.VMEM((_OB, _ROWS, _HW, _C_OUT), jnp.float32),
            pltpu.SemaphoreType.DMA((_XB, _S)),
            pltpu.SemaphoreType.DMA((_OB, _S)),
        ],
    )
    out = pl.pallas_call(
        _moe_body,
        grid_spec=grid_spec,
        out_shape=jax.ShapeDtypeStruct((_B, _HW, _C_OUT), jnp.float32),
        compiler_params=pltpu.CompilerParams(
            dimension_semantics=("arbitrary",),
        ),
    )(idx, w, xt, WcT, bc)
    return out.reshape(_B, _H, _W, _C_OUT).transpose(0, 3, 1, 2)


# speedup vs baseline: 1.0058x; 1.0058x over previous
import jax
import jax.numpy as jnp
from jax.experimental import pallas as pl
from jax.experimental.pallas import tpu as pltpu

_B, _C_IN, _C_OUT, _H, _W = 64, 384, 384, 24, 24
_HW = _H * _W
_E, _K = 8, 2

_ROWS = 4
_NG = _B // _ROWS
_S = 4
_RS = _ROWS // _S
_XB = 4
_OB = 3


def _moe_body(idx_ref, w_ref, x_hbm, WcT_ref, bc_ref, out_hbm,
              xb, ob, in_sems, out_sems):
    g = pl.program_id(0)

    def in_copy(gg, s):
        buf = jax.lax.rem(gg, _XB)
        return pltpu.make_async_copy(
            x_hbm.at[pl.ds(gg * _ROWS + s * _RS, _RS)],
            xb.at[buf, pl.ds(s * _RS, _RS)],
            in_sems.at[buf, s],
        )

    def out_copy(gg, s):
        buf = jax.lax.rem(gg, _OB)
        return pltpu.make_async_copy(
            ob.at[buf, pl.ds(s * _RS, _RS)],
            out_hbm.at[pl.ds(gg * _ROWS + s * _RS, _RS)],
            out_sems.at[buf, s],
        )

    @pl.when(g == 0)
    def _():
        for gg in range(_XB - 1):
            for s in range(_S):
                in_copy(gg, s).start()

    @pl.when(g < _NG - (_XB - 1))
    def _():
        for s in range(_S):
            in_copy(g + (_XB - 1), s).start()

    for s in range(_S):
        in_copy(g, s).wait()

    @pl.when(g >= _OB)
    def _():
        for s in range(_S):
            out_copy(g - _OB, s).wait()

    buf = jax.lax.rem(g, _XB)
    obuf = jax.lax.rem(g, _OB)
    for r in range(_ROWS):
        b = g * _ROWS + r
        i0 = idx_ref[0, b]
        i1 = idx_ref[1, b]
        w0 = w_ref[0, b]
        w1 = w_ref[1, b]
        W_eff = w0 * WcT_ref[i0] + w1 * WcT_ref[i1]
        out = jax.lax.dot_general(
            xb[buf, r], W_eff,
            (((1,), (1,)), ((), ())),
            preferred_element_type=jnp.float32)
        b_row = (w0 * bc_ref[pl.ds(i0, 1), :]
                 + w1 * bc_ref[pl.ds(i1, 1), :])
        ob[obuf, r] = out + b_row

    for s in range(_S):
        out_copy(g, s).start()

    @pl.when(g == _NG - 1)
    def _():
        for gg in range(_NG - _OB, _NG):
            for s in range(_S):
                out_copy(gg, s).wait()


def kernel(x, weights, indices, Wc, bc):
    xt = x.transpose(0, 2, 3, 1).reshape(_B, _HW, _C_IN)
    idx = indices.astype(jnp.int32).T
    w = weights.astype(jnp.float32).T
    WcT = Wc

    grid_spec = pltpu.PrefetchScalarGridSpec(
        num_scalar_prefetch=2,
        grid=(_NG,),
        in_specs=[
            pl.BlockSpec(memory_space=pltpu.MemorySpace.HBM),
            pl.BlockSpec((_E, _C_OUT, _C_IN), lambda b, *_: (0, 0, 0)),
            pl.BlockSpec((_E, _C_OUT), lambda b, *_: (0, 0)),
        ],
        out_specs=pl.BlockSpec(memory_space=pltpu.MemorySpace.HBM),
        scratch_shapes=[
            pltpu.VMEM((_XB, _ROWS, _HW, _C_IN), jnp.float32),
            pltpu.VMEM((_OB, _ROWS, _HW, _C_OUT), jnp.float32),
            pltpu.SemaphoreType.DMA((_XB, _S)),
            pltpu.SemaphoreType.DMA((_OB, _S)),
        ],
    )
    out = pl.pallas_call(
        _moe_body,
        grid_spec=grid_spec,
        out_shape=jax.ShapeDtypeStruct((_B, _HW, _C_OUT), jnp.float32),
        compiler_params=pltpu.CompilerParams(
            dimension_semantics=("arbitrary",),
        ),
    )(idx, w, xt, WcT, bc)
    return out.reshape(_B, _H, _W, _C_OUT).transpose(0, 3, 1, 2)
